# baseline (speedup 1.0000x reference)
import jax
import jax.numpy as jnp
from jax.experimental import pallas as pl
from jax.experimental.pallas import tpu as pltpu


def _fold_kernel(w_ref, h_ref, w1a_ref, w1b_ref, b1_ref, at_ref, bt_ref):
    dims = (((1,), (1,)), ((), ()))
    at_ref[...] = (jax.lax.dot_general(w1a_ref[...], w_ref[...], dims,
                                       preferred_element_type=jnp.float32)
                   + b1_ref[...])
    bt_ref[...] = jax.lax.dot_general(w1b_ref[...], h_ref[...], dims,
                                      preferred_element_type=jnp.float32)


def _score_kernel(au_ref, bv_ref, w2_ref, out_ref):
    h = jnp.maximum(au_ref[...] + bv_ref[...], 0.0)
    out_ref[...] = jnp.sum(w2_ref[...] * h, axis=0, keepdims=True)


def _ceil_div(a, b):
    return (a + b - 1) // b


@jax.jit
def _forward(x, W, H, w1, b1, w2):
    B = x.shape[0]
    NU, K = W.shape

    user_idx = x[:, 0].astype(jnp.int32)
    item_idx = x[:, 1].astype(jnp.int32)

    TN = 8192
    grid_fold = (_ceil_div(NU, TN),)
    At, Bt = pl.pallas_call(
        _fold_kernel,
        out_shape=(jax.ShapeDtypeStruct((K, NU), jnp.float32),
                   jax.ShapeDtypeStruct((K, NU), jnp.float32)),
        grid=grid_fold,
        in_specs=[
            pl.BlockSpec((TN, K), lambda i: (i, 0)),
            pl.BlockSpec((TN, K), lambda i: (i, 0)),
            pl.BlockSpec((K, K), lambda i: (0, 0)),
            pl.BlockSpec((K, K), lambda i: (0, 0)),
            pl.BlockSpec((K, 1), lambda i: (0, 0)),
        ],
        out_specs=(pl.BlockSpec((K, TN), lambda i: (0, i)),
                   pl.BlockSpec((K, TN), lambda i: (0, i))),
        compiler_params=pltpu.CompilerParams(
            dimension_semantics=("parallel",),
        ),
    )(W, H, w1[:, :K], w1[:, K:], b1.reshape(K, 1))

    au = At.at[:, user_idx].get(mode="promise_in_bounds")
    bv = Bt.at[:, item_idx].get(mode="promise_in_bounds")

    TB = min(8192, B)
    B_pad = _ceil_div(B, TB) * TB
    if B_pad != B:
        au = jnp.pad(au, ((0, 0), (0, B_pad - B)))
        bv = jnp.pad(bv, ((0, 0), (0, B_pad - B)))
    out = pl.pallas_call(
        _score_kernel,
        out_shape=jax.ShapeDtypeStruct((1, B_pad), jnp.float32),
        grid=(B_pad // TB,),
        in_specs=[
            pl.BlockSpec((K, TB), lambda i: (0, i)),
            pl.BlockSpec((K, TB), lambda i: (0, i)),
            pl.BlockSpec((K, 1), lambda i: (0, 0)),
        ],
        out_specs=pl.BlockSpec((1, TB), lambda i: (0, i)),
        compiler_params=pltpu.CompilerParams(
            dimension_semantics=("parallel",),
        ),
    )(au, bv, w2.reshape(K, 1))

    return out[0, :B].reshape(B, 1)


def kernel(x, W, H, w1, b1, w2):
    return _forward(x, W, H, w1, b1, w2)


# speedup vs baseline: 2.0430x; 2.0430x over previous
import jax
import jax.numpy as jnp
from jax.experimental import pallas as pl
from jax.experimental.pallas import tpu as pltpu


def _mlp_kernel(eu_ref, ev_ref, w1at_ref, w1bt_ref, b1_ref, w2_ref, out_ref):
    h = (jnp.dot(eu_ref[...], w1at_ref[...], preferred_element_type=jnp.float32)
         + jnp.dot(ev_ref[...], w1bt_ref[...], preferred_element_type=jnp.float32))
    h = jnp.maximum(h + b1_ref[...], 0.0)
    out_ref[...] = jax.lax.dot_general(
        w2_ref[...], h, (((1,), (1,)), ((), ())),
        preferred_element_type=jnp.float32)


def _round_up(n, m):
    return ((n + m - 1) // m) * m


@jax.jit
def _forward(x, W, H, w1, b1, w2):
    B = x.shape[0]
    K = W.shape[1]

    user_idx = x[:, 0].astype(jnp.int32)
    item_idx = x[:, 1].astype(jnp.int32)

    eu = W.at[user_idx].get(mode="promise_in_bounds")
    ev = H.at[item_idx].get(mode="promise_in_bounds")

    TB = min(8192, _round_up(B, 256) // 2)
    TB = max(256, (TB // 256) * 256)
    B_pad = _round_up(B, TB)
    if B_pad != B:
        eu = jnp.pad(eu, ((0, B_pad - B), (0, 0)))
        ev = jnp.pad(ev, ((0, B_pad - B), (0, 0)))

    out = pl.pallas_call(
        _mlp_kernel,
        out_shape=jax.ShapeDtypeStruct((1, B_pad), jnp.float32),
        grid=(B_pad // TB,),
        in_specs=[
            pl.BlockSpec((TB, K), lambda i: (i, 0)),
            pl.BlockSpec((TB, K), lambda i: (i, 0)),
            pl.BlockSpec((K, K), lambda i: (0, 0)),
            pl.BlockSpec((K, K), lambda i: (0, 0)),
            pl.BlockSpec((1, K), lambda i: (0, 0)),
            pl.BlockSpec((1, K), lambda i: (0, 0)),
        ],
        out_specs=pl.BlockSpec((1, TB), lambda i: (0, i)),
        compiler_params=pltpu.CompilerParams(
            dimension_semantics=("parallel",),
        ),
    )(eu, ev, w1[:, :K].T, w1[:, K:].T, b1.reshape(1, K), w2.reshape(1, K))

    return out[0, :B].reshape(B, 1)


def kernel(x, W, H, w1, b1, w2):
    return _forward(x, W, H, w1, b1, w2)
